# baseline (speedup 1.0000x reference)
import functools

import jax
import jax.numpy as jnp
from jax import lax
from jax.experimental import pallas as pl
from jax.experimental.pallas import tpu as pltpu
from jax.experimental.pallas import tpu_sc as plsc

D = 128
_C = 80
_L = 16


def _weights_body(c_ref, b_ref, w_ref):
    w_ref[...] = jnp.dot(c_ref[...], b_ref[...],
                         preferred_element_type=jnp.float32)


def _weights(comp, bases_flat):
    R = comp.shape[0]
    return pl.pallas_call(
        _weights_body,
        out_shape=jax.ShapeDtypeStruct((R, bases_flat.shape[1]), jnp.float32),
    )(comp, bases_flat)


def _relmm_body(x_ref, w_ref, h_ref):
    h_ref[...] = jnp.dot(x_ref[...], w_ref[...],
                         preferred_element_type=jnp.float32)


def _relmm(x, Wcat, bn, bc):
    N = x.shape[0]
    RD = Wcat.shape[1]
    return pl.pallas_call(
        _relmm_body,
        grid=(N // bn, RD // bc),
        in_specs=[
            pl.BlockSpec((bn, D), lambda i, j: (i, 0)),
            pl.BlockSpec((D, bc), lambda i, j: (0, j)),
        ],
        out_specs=pl.BlockSpec((bn, bc), lambda i, j: (i, j)),
        out_shape=jax.ShapeDtypeStruct((N, RD), jnp.float32),
    )(x, Wcat)


def _combine_body(x_ref, r_ref, b_ref, p0_ref, p1_ref, o_ref):
    acc = jnp.dot(x_ref[...], r_ref[...], preferred_element_type=jnp.float32)
    acc = acc + b_ref[...] + p0_ref[...] + p1_ref[...]
    o_ref[...] = jnp.maximum(acc, 0.0)


def _combine(x, root, bias2d, p0, p1, bn):
    N = x.shape[0]
    nt = N // bn
    return pl.pallas_call(
        _combine_body,
        grid=(nt,),
        in_specs=[
            pl.BlockSpec((bn, D), lambda i: (i, 0)),
            pl.BlockSpec((D, D), lambda i: (0, 0)),
            pl.BlockSpec((1, D), lambda i: (0, 0)),
            pl.BlockSpec((bn, D), lambda i: (i, 0)),
            pl.BlockSpec((bn, D), lambda i: (i, 0)),
        ],
        out_specs=pl.BlockSpec((bn, D), lambda i: (i, 0)),
        out_shape=jax.ShapeDtypeStruct((N, D), jnp.float32),
    )(x, root, bias2d, p0, p1)


def _edge_prep_body(N, R, E, NRp,
                    src_hbm, dst_hbm, et_hbm, gidx_hbm, scale_hbm,
                    cnt_sh,
                    ad0, ad1, ad2, ad3, at0, at1, at2, at3,
                    ax0, ax1, ax2, ax3, ones_v, zbuf,
                    bs0, bs1, bs2, bs3, bd0, bd1, bd2, bd3,
                    bt0, bt1, bt2, bt3, bx0, bx1, bx2, bx3,
                    bg0, bg1, bg2, bg3, bc0, bc1, bc2, bc3,
                    bo0, bo1, bo2, bo3,
                    ea0, ea1, ea2, ea3, sa0, sa1, sa2, sa3,
                    eb0, eb1, eb2, eb3, cg0, cg1, cg2, cg3,
                    ob0, ob1, ob2, ob3):
    cid = lax.axis_index("c")
    sid = lax.axis_index("s")
    wid = sid * 2 + cid
    ad = (ad0, ad1, ad2, ad3)
    at = (at0, at1, at2, at3)
    ax = (ax0, ax1, ax2, ax3)
    bs = (bs0, bs1, bs2, bs3)
    bd = (bd0, bd1, bd2, bd3)
    bt = (bt0, bt1, bt2, bt3)
    bx = (bx0, bx1, bx2, bx3)
    bg = (bg0, bg1, bg2, bg3)
    bc = (bc0, bc1, bc2, bc3)
    bo = (bo0, bo1, bo2, bo3)
    ea = (ea0, ea1, ea2, ea3)
    sa = (sa0, sa1, sa2, sa3)
    eb = (eb0, eb1, eb2, eb3)
    cg = (cg0, cg1, cg2, cg3)
    ob = (ob0, ob1, ob2, ob3)

    zn = zbuf.shape[0]

    def _zfill(k, _):
        zbuf[pl.ds(k * _L, _L)] = jnp.zeros((_L,), jnp.float32)
        return 0

    lax.fori_loop(0, zn // _L, _zfill, 0)
    per_tile = NRp // 16
    base_z = sid * per_tile

    def _zdma(k, _):
        pltpu.sync_copy(zbuf.at[pl.ds(0, zn)],
                        cnt_sh.at[pl.ds(base_z + k * zn, zn)])
        return 0

    lax.fori_loop(0, per_tile // zn, _zdma, 0)

    for k in range(_C // _L):
        ones_v[pl.ds(k * _L, _L)] = jnp.ones((_L,), jnp.float32)

    plsc.subcore_barrier()

    ec_sc = E // 16
    base_a = sid * ec_sc
    n_a = ec_sc // _C

    def _aload(b, c):
        off = base_a + c * _C
        pltpu.async_copy(dst_hbm.at[pl.ds(off, _C)], ad[b], ea[b])
        pltpu.async_copy(et_hbm.at[pl.ds(off, _C)], at[b], ea[b])

    def _awaitld(b):
        pltpu.make_async_copy(dst_hbm.at[pl.ds(0, _C)], ad[b], ea[b]).wait()
        pltpu.make_async_copy(et_hbm.at[pl.ds(0, _C)], at[b], ea[b]).wait()

    def _astep(c, b, do_sw=True, do_e=True):
        _awaitld(b)
        for k in range(_C // _L):
            sl = pl.ds(k * _L, _L)
            ax[b][sl] = ad[b][sl] * R + at[b][sl]
        if do_e:
            _aload((b + 2) % _NS, c + 2)
        pltpu.sync_copy(ones_v, cnt_sh.at[ax[b]], add=True)

    _aload(0, 0)
    _aload(1, 1)
    _astep(0, 0)
    _astep(1, 1)
    _astep(2, 2)
    _astep(3, 3)

    def _aloop(g, _):
        for b in range(_NS):
            _astep(g * _NS + b, b)
        return 0

    lax.fori_loop(1, n_a // _NS - 1, _aloop, 0)
    ca = n_a - _NS
    _astep(ca, ca % _NS)
    _astep(ca + 1, (ca + 1) % _NS)
    _astep(ca + 2, (ca + 2) % _NS, do_e=False)
    _astep(ca + 3, (ca + 3) % _NS, do_e=False)

    plsc.subcore_barrier()

    ec_w = E // 32
    base_b = wid * ec_w
    n_b = ec_w // _C

    def _bload(b, c):
        off = base_b + c * _C
        pltpu.async_copy(src_hbm.at[pl.ds(off, _C)], bs[b], eb[b])
        pltpu.async_copy(dst_hbm.at[pl.ds(off, _C)], bd[b], eb[b])
        pltpu.async_copy(et_hbm.at[pl.ds(off, _C)], bt[b], eb[b])

    def _bwaitld(b):
        pltpu.make_async_copy(src_hbm.at[pl.ds(0, _C)], bs[b], eb[b]).wait()
        pltpu.make_async_copy(dst_hbm.at[pl.ds(0, _C)], bd[b], eb[b]).wait()
        pltpu.make_async_copy(et_hbm.at[pl.ds(0, _C)], bt[b], eb[b]).wait()

    def _bstep(c, b, do_e=True):
        _bwaitld(b)
        for k in range(_C // _L):
            sl = pl.ds(k * _L, _L)
            bx[b][sl] = bd[b][sl] * R + bt[b][sl]
            bg[b][sl] = bs[b][sl] * R + bt[b][sl]
        if do_e:
            _bload((b + 2) % _NS, c + 2)
        pltpu.sync_copy(cnt_sh.at[bx[b]], bc[b])
        for k in range(_C // _L):
            sl = pl.ds(k * _L, _L)
            bo[b][sl] = 1.0 / bc[b][sl]
        off = base_b + c * _C
        pltpu.sync_copy(bg[b], gidx_hbm.at[pl.ds(off, _C)])
        pltpu.sync_copy(bo[b], scale_hbm.at[pl.ds(off, _C)])

    _bload(0, 0)
    _bload(1, 1)
    _bstep(0, 0)
    _bstep(1, 1)
    _bstep(2, 2)
    _bstep(3, 3)

    def _bloop(g, _):
        for b in range(_NS):
            _bstep(g * _NS + b, b)
        return 0

    lax.fori_loop(1, n_b // _NS - 1, _bloop, 0)
    cb = n_b - _NS
    _bstep(cb, cb % _NS)
    _bstep(cb + 1, (cb + 1) % _NS)
    _bstep(cb + 2, (cb + 2) % _NS, do_e=False)
    _bstep(cb + 3, (cb + 3) % _NS, do_e=False)


def _edge_prep(src, dst, et, N, R):
    E = src.shape[0]
    NRp = ((N * R + 16000 * 16 - 1) // (16000 * 16)) * (16000 * 16)
    mesh = plsc.VectorSubcoreMesh(core_axis_name="c", subcore_axis_name="s")
    kfn = pl.kernel(
        functools.partial(_edge_prep_body, N, R, E, NRp),
        out_type=(jax.ShapeDtypeStruct((E,), jnp.int32),
                  jax.ShapeDtypeStruct((E,), jnp.float32)),
        mesh=mesh,
        scratch_types=(
            [pltpu.VMEM_SHARED((NRp,), jnp.float32)]
            + [pltpu.VMEM((_C,), jnp.int32)] * (3 * _NS)
            + [pltpu.VMEM((_C,), jnp.float32),
               pltpu.VMEM((16000,), jnp.float32)]
            + [pltpu.VMEM((_C,), jnp.int32)] * (5 * _NS)
            + [pltpu.VMEM((_C,), jnp.float32)] * (2 * _NS)
            + [pltpu.SemaphoreType.DMA] * (5 * _NS)
        ),
    )
    return kfn(src, dst, et)


_NS = 4


def _aggregate_body(N, E,
                    h_hbm, gidx_hbm, dst_hbm, scale_hbm, p0_hbm, p1_hbm,
                    acc_sh,
                    rows0, rows1, rows2, rows3,
                    gv0, gv1, gv2, gv3, dv0, dv1, dv2, dv3,
                    sv0, sv1, sv2, sv3,
                    es0, es1, es2, es3, gs0, gs1, gs2, gs3,
                    ss0, ss1, ss2, ss3):
    cid = lax.axis_index("c")
    sid = lax.axis_index("s")
    wid = sid * 2 + cid
    rows = (rows0, rows1, rows2, rows3)
    gv = (gv0, gv1, gv2, gv3)
    dv = (dv0, dv1, dv2, dv3)
    sv = (sv0, sv1, sv2, sv3)
    esem = (es0, es1, es2, es3)
    gsem = (gs0, gs1, gs2, gs3)
    ssem = (ss0, ss1, ss2, ss3)

    rpt = (N // 16 // 8) * 8
    last = N - 15 * rpt
    base_r = sid * rpt

    def _zfill(i, _):
        for m in range(D // _L):
            rows0[i, pl.ds(m * _L, _L)] = jnp.zeros((_L,), jnp.float32)
        return 0

    lax.fori_loop(0, _C, _zfill, 0)

    nfull, rem = rpt // _C, rpt % _C
    for k in range(nfull):
        pltpu.sync_copy(rows0.at[pl.ds(0, _C)],
                        acc_sh.at[pl.ds(base_r + k * _C, _C)])
    if rem:
        pltpu.sync_copy(rows0.at[pl.ds(0, rem)],
                        acc_sh.at[pl.ds(base_r + nfull * _C, rem)])

    @pl.when(sid == 15)
    def _():
        extra = last - rpt
        off0 = 16 * rpt
        for k in range(extra // _C):
            pltpu.sync_copy(rows0.at[pl.ds(0, _C)],
                            acc_sh.at[pl.ds(off0 + k * _C, _C)])
        r2 = extra % _C
        if r2:
            pltpu.sync_copy(
                rows0.at[pl.ds(0, r2)],
                acc_sh.at[pl.ds(off0 + (extra // _C) * _C, r2)])

    plsc.subcore_barrier()

    ec_w = E // 32
    base_e = wid * ec_w
    nchunks = ec_w // _C

    def _eload(b, c):
        off = base_e + c * _C
        pltpu.async_copy(gidx_hbm.at[pl.ds(off, _C)], gv[b], esem[b])
        pltpu.async_copy(dst_hbm.at[pl.ds(off, _C)], dv[b], esem[b])
        pltpu.async_copy(scale_hbm.at[pl.ds(off, _C)], sv[b], esem[b])

    def _ewait(b):
        pltpu.make_async_copy(gidx_hbm.at[pl.ds(0, _C)], gv[b], esem[b]).wait()
        pltpu.make_async_copy(dst_hbm.at[pl.ds(0, _C)], dv[b], esem[b]).wait()
        pltpu.make_async_copy(scale_hbm.at[pl.ds(0, _C)], sv[b],
                              esem[b]).wait()

    def _gstart(b):
        pltpu.async_copy(h_hbm.at[gv[b]], rows[b], gsem[b])

    def _gwait(b):
        pltpu.make_async_copy(h_hbm.at[gv[b]], rows[b], gsem[b]).wait()

    def _sstart(b):
        pltpu.async_copy(rows[b], acc_sh.at[dv[b]], ssem[b], add=True)

    def _swait(b):
        pltpu.make_async_copy(rows[b], acc_sh.at[dv[b]], ssem[b]).wait()

    def _scale(b):
        def _blk(k, _):
            s16 = sv[b][pl.ds(k * _L, _L)]
            for t in range(_L):
                s = s16[t]
                e = k * _L + t
                for m in range(D // _L):
                    sl = pl.ds(m * _L, _L)
                    rows[b][e, sl] = rows[b][e, sl] * s
            return 0

        lax.fori_loop(0, _C // _L, _blk, 0)

    def _step(c, b, do_g2=True, do_sw=True, do_e3=True):
        b2, b3 = (b + 2) % _NS, (b + 3) % _NS
        if do_g2:
            _ewait(b2)
            _gstart(b2)
        _gwait(b)
        _scale(b)
        _sstart(b)
        if do_sw:
            _swait(b3)
        if do_e3:
            _eload(b3, c + 3)

    for b in range(3):
        _eload(b, b)
    for b in range(2):
        _ewait(b)
        _gstart(b)

    _step(0, 0, do_sw=False)
    _step(1, 1)
    _step(2, 2)
    _step(3, 3)

    def _gloop(g, _):
        for b in range(_NS):
            _step(g * _NS + b, b)
        return 0

    lax.fori_loop(1, nchunks // _NS - 1, _gloop, 0)

    c0 = nchunks - _NS
    _step(c0, c0 % _NS)
    _step(c0 + 1, (c0 + 1) % _NS, do_e3=False)
    _step(c0 + 2, (c0 + 2) % _NS, do_g2=False, do_e3=False)
    _step(c0 + 3, (c0 + 3) % _NS, do_g2=False, do_e3=False)
    _swait((c0 + 3) % _NS)

    plsc.subcore_barrier()

    @pl.when((cid == 0) & (sid < 15))
    def _():
        pltpu.sync_copy(acc_sh.at[pl.ds(base_r, rpt)],
                        p0_hbm.at[pl.ds(base_r, rpt)])

    @pl.when((cid == 0) & (sid == 15))
    def _():
        pltpu.sync_copy(acc_sh.at[pl.ds(15 * rpt, last)],
                        p0_hbm.at[pl.ds(15 * rpt, last)])

    @pl.when((cid == 1) & (sid < 15))
    def _():
        pltpu.sync_copy(acc_sh.at[pl.ds(base_r, rpt)],
                        p1_hbm.at[pl.ds(base_r, rpt)])

    @pl.when((cid == 1) & (sid == 15))
    def _():
        pltpu.sync_copy(acc_sh.at[pl.ds(15 * rpt, last)],
                        p1_hbm.at[pl.ds(15 * rpt, last)])


def _aggregate(h_flat, gidx, dst, scale, N):
    E = gidx.shape[0]
    mesh = plsc.VectorSubcoreMesh(core_axis_name="c", subcore_axis_name="s")
    kfn = pl.kernel(
        functools.partial(_aggregate_body, N, E),
        out_type=(jax.ShapeDtypeStruct((N, D), jnp.float32),
                  jax.ShapeDtypeStruct((N, D), jnp.float32)),
        mesh=mesh,
        scratch_types=(
            [pltpu.VMEM_SHARED((N, D), jnp.float32)]
            + [pltpu.VMEM((_C, D), jnp.float32)] * _NS
            + [pltpu.VMEM((_C,), jnp.int32)] * _NS
            + [pltpu.VMEM((_C,), jnp.int32)] * _NS
            + [pltpu.VMEM((_C,), jnp.float32)] * _NS
            + [pltpu.SemaphoreType.DMA] * (3 * _NS)
        ),
    )
    return kfn(h_flat, gidx, dst, scale)


def kernel(x, edge_index, edge_type, comp1, bases1, root1, bias1,
           comp2, bases2, root2, bias2):
    N, d_in = x.shape
    R, NB = comp1.shape
    E = edge_type.shape[0]

    src = edge_index[0].astype(jnp.int32)
    dst = edge_index[1].astype(jnp.int32)
    et = edge_type.astype(jnp.int32)

    bn = 1000 if N % 1000 == 0 else N // 10
    bc = 10 * D

    W1 = _weights(comp1, bases1.reshape(NB, d_in * D)).reshape(R, d_in, D)
    W2 = _weights(comp2, bases2.reshape(NB, D * D)).reshape(R, D, D)
    Wcat1 = W1.transpose(1, 0, 2).reshape(d_in, R * D)
    Wcat2 = W2.transpose(1, 0, 2).reshape(D, R * D)

    quant = 32 * _NS * _C
    Ep = ((E + quant - 1) // quant) * quant
    pad = Ep - E
    srcp = jnp.concatenate([src, jnp.zeros((pad,), jnp.int32)])
    etp = jnp.concatenate([et, jnp.zeros((pad,), jnp.int32)])
    dst_prep = jnp.concatenate([dst, jnp.full((pad,), N, jnp.int32)])
    dstp = jnp.concatenate([dst, jnp.zeros((pad,), jnp.int32)])

    gidx, scale = _edge_prep(srcp, dst_prep, etp, N, R)
    scale = jnp.concatenate([scale[:E], jnp.zeros((pad,), jnp.float32)])

    h1 = _relmm(x, Wcat1, bn, bc).reshape(N * R, D)
    p0, p1 = _aggregate(h1, gidx, dstp, scale, N)
    h = _combine(x, root1, bias1.reshape(1, D), p0, p1, bn)

    h2 = _relmm(h, Wcat2, bn, bc).reshape(N * R, D)
    q0, q1 = _aggregate(h2, gidx, dstp, scale, N)
    out = _combine(h, root2, bias2.reshape(1, D), q0, q1, bn)
    return out


# speedup vs baseline: 3.0032x; 3.0032x over previous
import functools

import jax
import jax.numpy as jnp
from jax import lax
from jax.experimental import pallas as pl
from jax.experimental.pallas import tpu as pltpu
from jax.experimental.pallas import tpu_sc as plsc

D = 128
_C = 80
_L = 16


def _weights_body(c_ref, b_ref, w_ref):
    w_ref[...] = jnp.dot(c_ref[...], b_ref[...],
                         preferred_element_type=jnp.float32)


def _weights(comp, bases_flat):
    R = comp.shape[0]
    return pl.pallas_call(
        _weights_body,
        out_shape=jax.ShapeDtypeStruct((R, bases_flat.shape[1]), jnp.float32),
    )(comp, bases_flat)


def _relmm_body(x_ref, w_ref, h_ref):
    h_ref[0] = jnp.dot(x_ref[...], w_ref[0],
                       preferred_element_type=jnp.float32)


def _relmm(x, W):
    N = x.shape[0]
    R = W.shape[0]
    return pl.pallas_call(
        _relmm_body,
        grid=(R,),
        in_specs=[
            pl.BlockSpec((N, D), lambda r: (0, 0)),
            pl.BlockSpec((1, D, D), lambda r: (r, 0, 0)),
        ],
        out_specs=pl.BlockSpec((1, N, D), lambda r: (r, 0, 0)),
        out_shape=jax.ShapeDtypeStruct((R, N, D), jnp.float32),
    )(x, W)


def _combine_body(x_ref, r_ref, b_ref, p0_ref, p1_ref, o_ref):
    acc = jnp.dot(x_ref[...], r_ref[...], preferred_element_type=jnp.float32)
    acc = acc + b_ref[...] + p0_ref[...] + p1_ref[...]
    o_ref[...] = jnp.maximum(acc, 0.0)


def _combine(x, root, bias2d, p0, p1, bn):
    N = x.shape[0]
    nt = N // bn
    return pl.pallas_call(
        _combine_body,
        grid=(nt,),
        in_specs=[
            pl.BlockSpec((bn, D), lambda i: (i, 0)),
            pl.BlockSpec((D, D), lambda i: (0, 0)),
            pl.BlockSpec((1, D), lambda i: (0, 0)),
            pl.BlockSpec((bn, D), lambda i: (i, 0)),
            pl.BlockSpec((bn, D), lambda i: (i, 0)),
        ],
        out_specs=pl.BlockSpec((bn, D), lambda i: (i, 0)),
        out_shape=jax.ShapeDtypeStruct((N, D), jnp.float32),
    )(x, root, bias2d, p0, p1)


def _edge_prep_body(N, R, E, NRp,
                    src_hbm, dst_hbm, et_hbm, gidx_hbm, scale_hbm,
                    cnt_sh,
                    ad0, ad1, ad2, ad3, at0, at1, at2, at3,
                    ax0, ax1, ax2, ax3, ones_v, zbuf,
                    bs0, bs1, bs2, bs3, bd0, bd1, bd2, bd3,
                    bt0, bt1, bt2, bt3, bx0, bx1, bx2, bx3,
                    bg0, bg1, bg2, bg3, bc0, bc1, bc2, bc3,
                    bo0, bo1, bo2, bo3,
                    ea0, ea1, ea2, ea3, sa0, sa1, sa2, sa3,
                    eb0, eb1, eb2, eb3, cg0, cg1, cg2, cg3,
                    ob0, ob1, ob2, ob3):
    cid = lax.axis_index("c")
    sid = lax.axis_index("s")
    wid = sid * 2 + cid
    ad = (ad0, ad1, ad2, ad3)
    at = (at0, at1, at2, at3)
    ax = (ax0, ax1, ax2, ax3)
    bs = (bs0, bs1, bs2, bs3)
    bd = (bd0, bd1, bd2, bd3)
    bt = (bt0, bt1, bt2, bt3)
    bx = (bx0, bx1, bx2, bx3)
    bg = (bg0, bg1, bg2, bg3)
    bc = (bc0, bc1, bc2, bc3)
    bo = (bo0, bo1, bo2, bo3)
    ea = (ea0, ea1, ea2, ea3)
    sa = (sa0, sa1, sa2, sa3)
    eb = (eb0, eb1, eb2, eb3)
    cg = (cg0, cg1, cg2, cg3)
    ob = (ob0, ob1, ob2, ob3)

    zn = zbuf.shape[0]

    def _zfill(k, _):
        zbuf[pl.ds(k * _L, _L)] = jnp.zeros((_L,), jnp.float32)
        return 0

    lax.fori_loop(0, zn // _L, _zfill, 0)
    per_tile = NRp // 16
    base_z = sid * per_tile

    def _zdma(k, _):
        pltpu.sync_copy(zbuf.at[pl.ds(0, zn)],
                        cnt_sh.at[pl.ds(base_z + k * zn, zn)])
        return 0

    lax.fori_loop(0, per_tile // zn, _zdma, 0)

    for k in range(_C // _L):
        ones_v[pl.ds(k * _L, _L)] = jnp.ones((_L,), jnp.float32)

    plsc.subcore_barrier()

    ec_sc = E // 16
    base_a = sid * ec_sc
    n_a = ec_sc // _C

    def _aload(b, c):
        off = base_a + c * _C
        pltpu.async_copy(dst_hbm.at[pl.ds(off, _C)], ad[b], ea[b])
        pltpu.async_copy(et_hbm.at[pl.ds(off, _C)], at[b], ea[b])

    def _awaitld(b):
        pltpu.make_async_copy(dst_hbm.at[pl.ds(0, _C)], ad[b], ea[b]).wait()
        pltpu.make_async_copy(et_hbm.at[pl.ds(0, _C)], at[b], ea[b]).wait()

    def _astep(c, b, do_sw=True, do_e=True):
        _awaitld(b)
        for k in range(_C // _L):
            sl = pl.ds(k * _L, _L)
            ax[b][sl] = ad[b][sl] * R + at[b][sl]
        if do_e:
            _aload((b + 2) % _NS, c + 2)
        pltpu.sync_copy(ones_v, cnt_sh.at[ax[b]], add=True)

    _aload(0, 0)
    _aload(1, 1)
    _astep(0, 0)
    _astep(1, 1)
    _astep(2, 2)
    _astep(3, 3)

    def _aloop(g, _):
        for b in range(_NS):
            _astep(g * _NS + b, b)
        return 0

    lax.fori_loop(1, n_a // _NS - 1, _aloop, 0)
    ca = n_a - _NS
    _astep(ca, ca % _NS)
    _astep(ca + 1, (ca + 1) % _NS)
    _astep(ca + 2, (ca + 2) % _NS, do_e=False)
    _astep(ca + 3, (ca + 3) % _NS, do_e=False)

    plsc.subcore_barrier()

    ec_w = E // 32
    base_b = wid * ec_w
    n_b = ec_w // _C

    def _bload(b, c):
        off = base_b + c * _C
        pltpu.async_copy(src_hbm.at[pl.ds(off, _C)], bs[b], eb[b])
        pltpu.async_copy(dst_hbm.at[pl.ds(off, _C)], bd[b], eb[b])
        pltpu.async_copy(et_hbm.at[pl.ds(off, _C)], bt[b], eb[b])

    def _bwaitld(b):
        pltpu.make_async_copy(src_hbm.at[pl.ds(0, _C)], bs[b], eb[b]).wait()
        pltpu.make_async_copy(dst_hbm.at[pl.ds(0, _C)], bd[b], eb[b]).wait()
        pltpu.make_async_copy(et_hbm.at[pl.ds(0, _C)], bt[b], eb[b]).wait()

    def _bstep(c, b, do_e=True):
        _bwaitld(b)
        for k in range(_C // _L):
            sl = pl.ds(k * _L, _L)
            bx[b][sl] = bd[b][sl] * R + bt[b][sl]
            bg[b][sl] = bt[b][sl] * N + bs[b][sl]
        if do_e:
            _bload((b + 2) % _NS, c + 2)
        pltpu.sync_copy(cnt_sh.at[bx[b]], bc[b])
        for k in range(_C // _L):
            sl = pl.ds(k * _L, _L)
            bo[b][sl] = 1.0 / bc[b][sl]
        off = base_b + c * _C
        pltpu.sync_copy(bg[b], gidx_hbm.at[pl.ds(off, _C)])
        pltpu.sync_copy(bo[b], scale_hbm.at[pl.ds(off, _C)])

    _bload(0, 0)
    _bload(1, 1)
    _bstep(0, 0)
    _bstep(1, 1)
    _bstep(2, 2)
    _bstep(3, 3)

    def _bloop(g, _):
        for b in range(_NS):
            _bstep(g * _NS + b, b)
        return 0

    lax.fori_loop(1, n_b // _NS - 1, _bloop, 0)
    cb = n_b - _NS
    _bstep(cb, cb % _NS)
    _bstep(cb + 1, (cb + 1) % _NS)
    _bstep(cb + 2, (cb + 2) % _NS, do_e=False)
    _bstep(cb + 3, (cb + 3) % _NS, do_e=False)


def _edge_prep(src, dst, et, N, R):
    E = src.shape[0]
    NRp = ((N * R + 16000 * 16 - 1) // (16000 * 16)) * (16000 * 16)
    mesh = plsc.VectorSubcoreMesh(core_axis_name="c", subcore_axis_name="s")
    kfn = pl.kernel(
        functools.partial(_edge_prep_body, N, R, E, NRp),
        out_type=(jax.ShapeDtypeStruct((E,), jnp.int32),
                  jax.ShapeDtypeStruct((E,), jnp.float32)),
        mesh=mesh,
        scratch_types=(
            [pltpu.VMEM_SHARED((NRp,), jnp.float32)]
            + [pltpu.VMEM((_C,), jnp.int32)] * (3 * _NS)
            + [pltpu.VMEM((_C,), jnp.float32),
               pltpu.VMEM((16000,), jnp.float32)]
            + [pltpu.VMEM((_C,), jnp.int32)] * (5 * _NS)
            + [pltpu.VMEM((_C,), jnp.float32)] * (2 * _NS)
            + [pltpu.SemaphoreType.DMA] * (5 * _NS)
        ),
    )
    return kfn(src, dst, et)


_NS = 4


def _aggregate_body(N, E,
                    h_hbm, gidx_hbm, dst_hbm, scale_hbm, p0_hbm, p1_hbm,
                    acc_sh,
                    rows0, rows1, rows2, rows3,
                    gv0, gv1, gv2, gv3, dv0, dv1, dv2, dv3,
                    sv0, sv1, sv2, sv3,
                    es0, es1, es2, es3, gs0, gs1, gs2, gs3,
                    ss0, ss1, ss2, ss3):
    cid = lax.axis_index("c")
    sid = lax.axis_index("s")
    wid = sid * 2 + cid
    rows = (rows0, rows1, rows2, rows3)
    gv = (gv0, gv1, gv2, gv3)
    dv = (dv0, dv1, dv2, dv3)
    sv = (sv0, sv1, sv2, sv3)
    esem = (es0, es1, es2, es3)
    gsem = (gs0, gs1, gs2, gs3)
    ssem = (ss0, ss1, ss2, ss3)

    rpt = (N // 16 // 8) * 8
    last = N - 15 * rpt
    base_r = sid * rpt

    def _zfill(i, _):
        for m in range(D // _L):
            rows0[i, pl.ds(m * _L, _L)] = jnp.zeros((_L,), jnp.float32)
        return 0

    lax.fori_loop(0, _C, _zfill, 0)

    nfull, rem = rpt // _C, rpt % _C
    for k in range(nfull):
        pltpu.sync_copy(rows0.at[pl.ds(0, _C)],
                        acc_sh.at[pl.ds(base_r + k * _C, _C)])
    if rem:
        pltpu.sync_copy(rows0.at[pl.ds(0, rem)],
                        acc_sh.at[pl.ds(base_r + nfull * _C, rem)])

    @pl.when(sid == 15)
    def _():
        extra = last - rpt
        off0 = 16 * rpt
        for k in range(extra // _C):
            pltpu.sync_copy(rows0.at[pl.ds(0, _C)],
                            acc_sh.at[pl.ds(off0 + k * _C, _C)])
        r2 = extra % _C
        if r2:
            pltpu.sync_copy(
                rows0.at[pl.ds(0, r2)],
                acc_sh.at[pl.ds(off0 + (extra // _C) * _C, r2)])

    plsc.subcore_barrier()

    ec_w = E // 32
    base_e = wid * ec_w
    nchunks = ec_w // _C

    def _eload(b, c):
        off = base_e + c * _C
        pltpu.async_copy(gidx_hbm.at[pl.ds(off, _C)], gv[b], esem[b])
        pltpu.async_copy(dst_hbm.at[pl.ds(off, _C)], dv[b], esem[b])
        pltpu.async_copy(scale_hbm.at[pl.ds(off, _C)], sv[b], esem[b])

    def _ewait(b):
        pltpu.make_async_copy(gidx_hbm.at[pl.ds(0, _C)], gv[b], esem[b]).wait()
        pltpu.make_async_copy(dst_hbm.at[pl.ds(0, _C)], dv[b], esem[b]).wait()
        pltpu.make_async_copy(scale_hbm.at[pl.ds(0, _C)], sv[b],
                              esem[b]).wait()

    def _gstart(b):
        pltpu.async_copy(h_hbm.at[gv[b]], rows[b], gsem[b])

    def _gwait(b):
        pltpu.make_async_copy(h_hbm.at[gv[b]], rows[b], gsem[b]).wait()

    def _sstart(b):
        pltpu.async_copy(rows[b], acc_sh.at[dv[b]], ssem[b], add=True)

    def _swait(b):
        pltpu.make_async_copy(rows[b], acc_sh.at[dv[b]], ssem[b]).wait()

    def _scale(b):
        def _blk(k, _):
            s16 = sv[b][pl.ds(k * _L, _L)]
            for t in range(_L):
                s = s16[t]
                e = k * _L + t
                for m in range(D // _L):
                    sl = pl.ds(m * _L, _L)
                    rows[b][e, sl] = rows[b][e, sl] * s
            return 0

        lax.fori_loop(0, _C // _L, _blk, 0)

    def _step(c, b, do_g2=True, do_sw=True, do_e3=True):
        b2, b3 = (b + 2) % _NS, (b + 3) % _NS
        if do_g2:
            _ewait(b2)
            _gstart(b2)
        _gwait(b)
        _scale(b)
        _sstart(b)
        if do_sw:
            _swait(b3)
        if do_e3:
            _eload(b3, c + 3)

    for b in range(3):
        _eload(b, b)
    for b in range(2):
        _ewait(b)
        _gstart(b)

    _step(0, 0, do_sw=False)
    _step(1, 1)
    _step(2, 2)
    _step(3, 3)

    def _gloop(g, _):
        for b in range(_NS):
            _step(g * _NS + b, b)
        return 0

    lax.fori_loop(1, nchunks // _NS - 1, _gloop, 0)

    c0 = nchunks - _NS
    _step(c0, c0 % _NS)
    _step(c0 + 1, (c0 + 1) % _NS, do_e3=False)
    _step(c0 + 2, (c0 + 2) % _NS, do_g2=False, do_e3=False)
    _step(c0 + 3, (c0 + 3) % _NS, do_g2=False, do_e3=False)
    _swait((c0 + 3) % _NS)

    plsc.subcore_barrier()

    @pl.when((cid == 0) & (sid < 15))
    def _():
        pltpu.sync_copy(acc_sh.at[pl.ds(base_r, rpt)],
                        p0_hbm.at[pl.ds(base_r, rpt)])

    @pl.when((cid == 0) & (sid == 15))
    def _():
        pltpu.sync_copy(acc_sh.at[pl.ds(15 * rpt, last)],
                        p0_hbm.at[pl.ds(15 * rpt, last)])

    @pl.when((cid == 1) & (sid < 15))
    def _():
        pltpu.sync_copy(acc_sh.at[pl.ds(base_r, rpt)],
                        p1_hbm.at[pl.ds(base_r, rpt)])

    @pl.when((cid == 1) & (sid == 15))
    def _():
        pltpu.sync_copy(acc_sh.at[pl.ds(15 * rpt, last)],
                        p1_hbm.at[pl.ds(15 * rpt, last)])


def _aggregate(h_flat, gidx, dst, scale, N):
    E = gidx.shape[0]
    mesh = plsc.VectorSubcoreMesh(core_axis_name="c", subcore_axis_name="s")
    kfn = pl.kernel(
        functools.partial(_aggregate_body, N, E),
        out_type=(jax.ShapeDtypeStruct((N, D), jnp.float32),
                  jax.ShapeDtypeStruct((N, D), jnp.float32)),
        mesh=mesh,
        scratch_types=(
            [pltpu.VMEM_SHARED((N, D), jnp.float32)]
            + [pltpu.VMEM((_C, D), jnp.float32)] * _NS
            + [pltpu.VMEM((_C,), jnp.int32)] * _NS
            + [pltpu.VMEM((_C,), jnp.int32)] * _NS
            + [pltpu.VMEM((_C,), jnp.float32)] * _NS
            + [pltpu.SemaphoreType.DMA] * (3 * _NS)
        ),
    )
    return kfn(h_flat, gidx, dst, scale)


def kernel(x, edge_index, edge_type, comp1, bases1, root1, bias1,
           comp2, bases2, root2, bias2):
    N, d_in = x.shape
    R, NB = comp1.shape
    E = edge_type.shape[0]

    src = edge_index[0].astype(jnp.int32)
    dst = edge_index[1].astype(jnp.int32)
    et = edge_type.astype(jnp.int32)

    bn = 1000 if N % 1000 == 0 else N // 10

    W1 = _weights(comp1, bases1.reshape(NB, d_in * D)).reshape(R, d_in, D)
    W2 = _weights(comp2, bases2.reshape(NB, D * D)).reshape(R, D, D)

    quant = 32 * _NS * _C
    Ep = ((E + quant - 1) // quant) * quant
    pad = Ep - E
    pidx = jnp.arange(pad, dtype=jnp.int32)
    srcp = jnp.concatenate([src, jnp.zeros((pad,), jnp.int32)])
    etp = jnp.concatenate([et, pidx % R])
    dst_prep = jnp.concatenate([dst, N + pidx // R])
    dstp = jnp.concatenate([dst, pidx % N])
    gidx, scale = _edge_prep(srcp, dst_prep, etp, N, R)
    scale = jnp.concatenate([scale[:E], jnp.zeros((pad,), jnp.float32)])

    h1 = _relmm(x, W1).reshape(R * N, D)
    p0, p1 = _aggregate(h1, gidx, dstp, scale, N)
    h = _combine(x, root1, bias1.reshape(1, D), p0, p1, bn)

    h2 = _relmm(h, W2).reshape(R * N, D)
    q0, q1 = _aggregate(h2, gidx, dstp, scale, N)
    out = _combine(h, root2, bias2.reshape(1, D), q0, q1, bn)
    return out
